# baseline (speedup 1.0000x reference)
import functools

import numpy as np
import jax
import jax.numpy as jnp
from jax import lax
from jax.experimental import pallas as pl
from jax.experimental.pallas import tpu as pltpu
from jax.experimental.pallas import tpu_sc as plsc

_DEGREE_RES = 10
_NUM_ROT = 36
_IM_SIZE = 64
_B = 16
_N = 16384
_NC = 2
_NS = 16
_NW = _NC * _NS
_RPW = _B * _NUM_ROT // _NW
_NCHUNK = _N // 16

_MAGIC = np.float32(12582912.0)


def _bf16(v):
    return np.asarray(jnp.asarray(v, jnp.float32).astype(jnp.bfloat16),
                      jnp.bfloat16).astype(np.float32)


def _make_tables():
    ang = np.radians(np.arange(_NUM_ROT) * _DEGREE_RES)
    c = _bf16(np.cos(ang).astype(np.float32))
    s = _bf16(np.sin(ang).astype(np.float32))
    splat = lambda v: np.repeat(v.astype(np.float32), 16)
    return np.concatenate(
        [splat(c * np.float32(16.0)), splat(s * np.float32(16.0)),
         splat(s / np.float32(10.0)), splat(c / np.float32(10.0))])


_TBL = jnp.asarray(_make_tables())


def _bf16_round(a):
    v = jax.lax.bitcast_convert_type(a, jnp.uint32)
    r = (v + jnp.uint32(0x7FFF) + ((v >> 16) & jnp.uint32(1))) & jnp.uint32(
        0xFFFF0000)
    return jax.lax.bitcast_convert_type(r, jnp.float32)


_NROT_PASS = 9


def _body(xyzt_hbm, tbl_hbm, out_hbm, xv, zv, pyvf, tblv, *imgs_sems):
    imgs = imgs_sems[:_NROT_PASS]
    sems = imgs_sems[_NROT_PASS:]
    wid = lax.axis_index("s") * _NC + lax.axis_index("c")
    b = wid // 2
    half = wid - 2 * b
    pair0 = b * _NUM_ROT + half * _RPW

    pltpu.sync_copy(xyzt_hbm.at[0, b], xv)
    pltpu.sync_copy(xyzt_hbm.at[1, b], pyvf)
    pltpu.sync_copy(xyzt_hbm.at[2, b], zv)
    pltpu.sync_copy(tbl_hbm, tblv)

    zero16f = jnp.zeros((16,), jnp.float32)

    def py_body(i, carry):
        base = i * 64
        for k in range(4):
            sl = pl.ds(base + k * 16, 16)
            y16 = pyvf[sl]
            w = ((y16 + 2.0) * 16.0 + _MAGIC) - _MAGIC
            py = w.astype(jnp.int32)
            oky = py.astype(jnp.uint32) < jnp.uint32(64)
            pyvf[sl] = plsc.bitcast(jnp.where(oky, py * 64, -1048576),
                                    jnp.float32)
        return carry

    lax.fori_loop(0, _NCHUNK // 4, py_body, 0)

    def outer(j, carry):
        r = half * _RPW + j * _NROT_PASS

        @pl.when(j > 0)
        def _():
            for img, sem in zip(imgs, sems):
                pltpu.make_async_copy(img, out_hbm.at[pair0], sem).wait()

        consts = []
        for m in range(_NROT_PASS):
            off = (r + m) * 16
            consts.append((imgs[m],
                           tblv[pl.ds(off, 16)],
                           tblv[pl.ds(576 + off, 16)],
                           tblv[pl.ds(1152 + off, 16)],
                           tblv[pl.ds(1728 + off, 16)]))

        def z_body(i, zc):
            sl = pl.ds(i * 16, 16)
            for img in imgs:
                img[sl] = zero16f
            return zc

        lax.fori_loop(0, 256, z_body, 0)

        def chunk(i, cc):
            base = i * 32
            for k in range(2):
                sl = pl.ds(base + k * 16, 16)
                xx = xv[sl]
                zz = zv[sl]
                pyv16 = plsc.bitcast(pyvf[sl], jnp.int32)
                for img, cb, sb, sd, cd in consts:
                    w = ((xx * cb - zz * sb + 32.0) + _MAGIC) - _MAGIC
                    px = w.astype(jnp.int32)
                    okx = px.astype(jnp.uint32) < jnp.uint32(64)
                    idx0 = pyv16 + px
                    ok = okx & (idx0 >= 0)
                    idx = jnp.where(ok, idx0, 0)
                    val = xx * sd + zz * cd
                    plsc.store_scatter(img, [idx], val)
            return cc

        lax.fori_loop(0, _NCHUNK // 2, chunk, 0)

        for m, (img, sem) in enumerate(zip(imgs, sems)):
            pltpu.async_copy(img, out_hbm.at[pair0 + j * _NROT_PASS + m], sem)
        return carry

    lax.fori_loop(0, _RPW // _NROT_PASS, outer, 0)
    for img, sem in zip(imgs, sems):
        pltpu.make_async_copy(img, out_hbm.at[pair0], sem).wait()


@functools.partial(jax.jit, static_argnames=())
def kernel(xyz):
    xyzt = jnp.transpose(_bf16_round(xyz), (2, 0, 1))
    call = pl.kernel(
        _body,
        out_type=jax.ShapeDtypeStruct((_B * _NUM_ROT, _IM_SIZE * _IM_SIZE),
                                      jnp.float32),
        mesh=plsc.VectorSubcoreMesh(core_axis_name="c", subcore_axis_name="s"),
        compiler_params=pltpu.CompilerParams(needs_layout_passes=False),
        scratch_types=[
            pltpu.VMEM((_N,), jnp.float32),
            pltpu.VMEM((_N,), jnp.float32),
            pltpu.VMEM((_N,), jnp.float32),
            pltpu.VMEM((4 * _NUM_ROT * 16,), jnp.float32),
        ] + [pltpu.VMEM((_IM_SIZE * _IM_SIZE,), jnp.float32)
             for _ in range(_NROT_PASS)]
          + [pltpu.SemaphoreType.DMA for _ in range(_NROT_PASS)],
    )
    out = call(xyzt, _TBL)
    return out.reshape(_B, _NUM_ROT, _IM_SIZE, _IM_SIZE)


# speedup vs baseline: 1.0103x; 1.0103x over previous
import functools

import numpy as np
import jax
import jax.numpy as jnp
from jax import lax
from jax.experimental import pallas as pl
from jax.experimental.pallas import tpu as pltpu
from jax.experimental.pallas import tpu_sc as plsc

_DEGREE_RES = 10
_NUM_ROT = 36
_IM_SIZE = 64
_B = 16
_N = 16384
_NC = 2
_NS = 16
_NW = _NC * _NS
_RPW = _B * _NUM_ROT // _NW
_NCHUNK = _N // 16

_MAGIC = np.float32(12582912.0)


def _bf16(v):
    return np.asarray(jnp.asarray(v, jnp.float32).astype(jnp.bfloat16),
                      jnp.bfloat16).astype(np.float32)


def _make_tables():
    ang = np.radians(np.arange(_NUM_ROT) * _DEGREE_RES)
    c = _bf16(np.cos(ang).astype(np.float32))
    s = _bf16(np.sin(ang).astype(np.float32))
    splat = lambda v: np.repeat(v.astype(np.float32), 16)
    return np.concatenate(
        [splat(c * np.float32(16.0)), splat(s * np.float32(16.0)),
         splat(s / np.float32(10.0)), splat(c / np.float32(10.0))])


_TBL = jnp.asarray(_make_tables())


def _bf16_round(a):
    v = jax.lax.bitcast_convert_type(a, jnp.uint32)
    r = (v + jnp.uint32(0x7FFF) + ((v >> 16) & jnp.uint32(1))) & jnp.uint32(
        0xFFFF0000)
    return jax.lax.bitcast_convert_type(r, jnp.float32)


_NROT_PASS = 9


def _body(xyzt_hbm, tbl_hbm, out_hbm, xv, yv, zv, pyvv, tblv, *imgs_sems):
    imgs = imgs_sems[:_NROT_PASS]
    sems = imgs_sems[_NROT_PASS:]
    wid = lax.axis_index("s") * _NC + lax.axis_index("c")
    b = wid // 2
    half = wid - 2 * b
    pair0 = b * _NUM_ROT + half * _RPW

    pltpu.sync_copy(xyzt_hbm.at[0, b], xv)
    pltpu.sync_copy(xyzt_hbm.at[1, b], yv)
    pltpu.sync_copy(xyzt_hbm.at[2, b], zv)
    pltpu.sync_copy(tbl_hbm, tblv)

    zero16f = jnp.zeros((16,), jnp.float32)

    def py_body(i, carry):
        base = i * 64
        for k in range(4):
            sl = pl.ds(base + k * 16, 16)
            y16 = yv[sl]
            w = ((y16 + 2.0) * 16.0 + _MAGIC) - _MAGIC
            py = w.astype(jnp.int32)
            oky = py.astype(jnp.uint32) < jnp.uint32(64)
            pyvv[sl] = jnp.where(oky, py * 64, -1048576)
        return carry

    lax.fori_loop(0, _NCHUNK // 4, py_body, 0)

    def outer(j, carry):
        r = half * _RPW + j * _NROT_PASS

        @pl.when(j > 0)
        def _():
            for img, sem in zip(imgs, sems):
                pltpu.make_async_copy(img, out_hbm.at[pair0], sem).wait()

        consts = []
        for m in range(_NROT_PASS):
            off = (r + m) * 16
            consts.append((imgs[m],
                           tblv[pl.ds(off, 16)],
                           tblv[pl.ds(576 + off, 16)],
                           tblv[pl.ds(1152 + off, 16)],
                           tblv[pl.ds(1728 + off, 16)]))

        def z_body(i, zc):
            sl = pl.ds(i * 16, 16)
            for img in imgs:
                img[sl] = zero16f
            return zc

        lax.fori_loop(0, 256, z_body, 0)

        def chunk(i, cc):
            base = i * 32
            for k in range(2):
                sl = pl.ds(base + k * 16, 16)
                xx = xv[sl]
                zz = zv[sl]
                pyv16 = pyvv[sl]
                for img, cb, sb, sd, cd in consts:
                    w = ((xx * cb - zz * sb + 32.0) + _MAGIC) - _MAGIC
                    px = w.astype(jnp.int32)
                    okx = px.astype(jnp.uint32) < jnp.uint32(64)
                    idx0 = pyv16 + px
                    ok = okx & (idx0 >= 0)
                    idx = jnp.where(ok, idx0, 0)
                    val = xx * sd + zz * cd
                    plsc.store_scatter(img, [idx], val)
            return cc

        lax.fori_loop(0, _NCHUNK // 2, chunk, 0)

        for m, (img, sem) in enumerate(zip(imgs, sems)):
            pltpu.async_copy(img, out_hbm.at[pair0 + j * _NROT_PASS + m], sem)
        return carry

    lax.fori_loop(0, _RPW // _NROT_PASS, outer, 0)
    for img, sem in zip(imgs, sems):
        pltpu.make_async_copy(img, out_hbm.at[pair0], sem).wait()


@functools.partial(jax.jit, static_argnames=())
def kernel(xyz):
    xyzt = jnp.transpose(_bf16_round(xyz), (2, 0, 1))
    call = pl.kernel(
        _body,
        out_type=jax.ShapeDtypeStruct((_B * _NUM_ROT, _IM_SIZE * _IM_SIZE),
                                      jnp.float32),
        mesh=plsc.VectorSubcoreMesh(core_axis_name="c", subcore_axis_name="s"),
        compiler_params=pltpu.CompilerParams(needs_layout_passes=False),
        scratch_types=[
            pltpu.VMEM((_N,), jnp.float32),
            pltpu.VMEM((_N,), jnp.float32),
            pltpu.VMEM((_N,), jnp.float32),
            pltpu.VMEM((_N,), jnp.int32),
            pltpu.VMEM((4 * _NUM_ROT * 16,), jnp.float32),
        ] + [pltpu.VMEM((_IM_SIZE * _IM_SIZE,), jnp.float32)
             for _ in range(_NROT_PASS)]
          + [pltpu.SemaphoreType.DMA for _ in range(_NROT_PASS)],
    )
    out = call(xyzt, _TBL)
    return out.reshape(_B, _NUM_ROT, _IM_SIZE, _IM_SIZE)
